# baseline (speedup 1.0000x reference)
import functools

import jax
import jax.numpy as jnp
from jax import lax
from jax.experimental import pallas as pl
from jax.experimental.pallas import tpu as pltpu
from jax.experimental.pallas import tpu_sc as plsc

N = 10000
D = 128
E = 320000
EPS = 1e-5

NC = 2
NS = 16
NW = NC * NS

B = 128
EPT = E // NW
CHUNKS = 4 * (-(-EPT // (4 * B)))
HALF = CHUNKS // 2
HPAIRS = HALF // 2
EPT_PAD = CHUNKS * B
E_PAD = EPT_PAD * NW

N_PAD = 10240
RPT = N_PAD // NS

_mesh = plsc.VectorSubcoreMesh(core_axis_name="c", subcore_axis_name="s")


@functools.partial(
    pl.kernel,
    mesh=_mesh,
    out_type=jax.ShapeDtypeStruct((NC, N_PAD), jnp.float32),
    scratch_types=[
        pltpu.VMEM((CHUNKS, B), jnp.int32),
        pltpu.VMEM((B,), jnp.float32),
        pltpu.VMEM_SHARED((N_PAD,), jnp.float32),
    ],
)
def _deg_kernel(dst_hbm, zero_hbm, out_hbm, dst_v, ones_v, shared):
    cid = lax.axis_index("c")
    sid = lax.axis_index("s")
    wid = sid * NC + cid
    pltpu.sync_copy(dst_hbm.at[wid], dst_v)
    for j in range(B // 16):
        ones_v[pl.ds(j * 16, 16)] = jnp.ones((16,), jnp.float32)
    pltpu.sync_copy(zero_hbm, shared.at[pl.ds(sid * RPT, RPT)])
    plsc.subcore_barrier()

    def body(c, carry):
        pltpu.sync_copy(ones_v, shared.at[dst_v.at[c]], add=True)
        return carry

    lax.fori_loop(0, CHUNKS, body, 0)
    plsc.subcore_barrier()
    pltpu.sync_copy(shared.at[pl.ds(sid * RPT, RPT)],
                    out_hbm.at[cid, pl.ds(sid * RPT, RPT)])


@functools.partial(
    pl.kernel,
    mesh=_mesh,
    out_type=jax.ShapeDtypeStruct((NC, N_PAD, D), jnp.float32),
    scratch_types=[
        pltpu.VMEM((HALF, B), jnp.int32),
        pltpu.VMEM((HALF, B), jnp.int32),
        pltpu.VMEM((B, D), jnp.float32),
        pltpu.VMEM((B, D), jnp.float32),
        pltpu.VMEM_SHARED((N_PAD, D), jnp.float32),
        pltpu.SemaphoreType.DMA,
    ],
)
def _scatter_kernel(hs_hbm, src_hbm, dst_hbm, zero_hbm, out_hbm,
                    src_v, dst_v, rows0, rows1, shared, sem):
    cid = lax.axis_index("c")
    sid = lax.axis_index("s")
    wid = sid * NC + cid
    pltpu.sync_copy(src_hbm.at[wid, 0], src_v)
    pltpu.sync_copy(dst_hbm.at[wid, 0], dst_v)
    pltpu.async_copy(hs_hbm.at[src_v.at[0]], rows0, sem)
    pltpu.sync_copy(zero_hbm, shared.at[pl.ds(sid * RPT, RPT)])
    plsc.subcore_barrier()

    def _drain(buf):
        pltpu.make_async_copy(hs_hbm.at[pl.ds(0, B)], buf, sem).wait()

    def _pair(c0, fire_next):
        c1 = c0 + 1
        _drain(rows0)
        pltpu.async_copy(hs_hbm.at[src_v.at[c1]], rows1, sem)
        pltpu.sync_copy(rows0, shared.at[dst_v.at[c0]], add=True)
        _drain(rows1)
        if fire_next:
            pltpu.async_copy(hs_hbm.at[src_v.at[c0 + 2]], rows0, sem)
        pltpu.sync_copy(rows1, shared.at[dst_v.at[c1]], add=True)

    def body(g, carry):
        _pair(2 * g, True)
        return carry

    lax.fori_loop(0, HPAIRS - 1, body, 0)
    _pair(2 * (HPAIRS - 1), False)
    pltpu.sync_copy(src_hbm.at[wid, 1], src_v)
    pltpu.sync_copy(dst_hbm.at[wid, 1], dst_v)
    pltpu.async_copy(hs_hbm.at[src_v.at[0]], rows0, sem)
    lax.fori_loop(0, HPAIRS - 1, body, 0)
    _pair(2 * (HPAIRS - 1), False)
    plsc.subcore_barrier()
    pltpu.sync_copy(shared.at[pl.ds(sid * RPT, RPT)],
                    out_hbm.at[cid, pl.ds(sid * RPT, RPT)])


def _dense1_body(x_ref, w_ref, d_ref, hs_ref, dinv_ref):
    deg = 1.0 + d_ref[0, :, :] + d_ref[1, :, :]
    dv = lax.rsqrt(deg)
    h = jnp.dot(x_ref[...], w_ref[...], preferred_element_type=jnp.float32)
    hs_ref[...] = h * dv
    dinv_ref[...] = dv


_dense1 = pl.pallas_call(
    _dense1_body,
    out_shape=(jax.ShapeDtypeStruct((N, D), jnp.float32),
               jax.ShapeDtypeStruct((N, 1), jnp.float32)),
)


def _dense2_body(s_ref, hs_ref, dinv_ref, b_ref, g_ref, bt_ref, o_ref):
    agg = dinv_ref[...] * (s_ref[0, :, :] + s_ref[1, :, :] + hs_ref[...])
    act = jnp.tanh(agg + b_ref[...])
    mean = jnp.mean(act, axis=0, keepdims=True)
    cent = act - mean
    var = jnp.mean(cent * cent, axis=0, keepdims=True)
    o_ref[...] = g_ref[...] * cent * lax.rsqrt(var + EPS) + bt_ref[...]


_dense2 = pl.pallas_call(
    _dense2_body,
    out_shape=jax.ShapeDtypeStruct((N, D), jnp.float32),
)


def kernel(x, edge_index, W, b, gamma, beta):
    src = edge_index[0].astype(jnp.int32)
    dst = edge_index[1].astype(jnp.int32)
    pad = E_PAD - E
    pad_dst = N + (jnp.arange(pad, dtype=jnp.int32) % (N_PAD - N))
    src = jnp.concatenate([src, jnp.zeros((pad,), jnp.int32)]).reshape(NW, CHUNKS, B)
    dst = jnp.concatenate([dst, pad_dst]).reshape(NW, CHUNKS, B)
    src2 = src.reshape(NW, 2, HALF, B)
    dst2 = dst.reshape(NW, 2, HALF, B)

    zero_deg = jnp.zeros((RPT,), jnp.float32)
    zero_row = jnp.zeros((RPT, D), jnp.float32)

    cnt = _deg_kernel(dst, zero_deg)
    hs, dinv = _dense1(x, W, cnt[:, :N].reshape(NC, N, 1))
    S2 = _scatter_kernel(hs, src2, dst2, zero_row)
    out = _dense2(S2[:, :N, :], hs, dinv,
                  b.reshape(1, D), gamma.reshape(1, D), beta.reshape(1, D))
    return out
